# baseline (speedup 1.0000x reference)
import functools

import jax
import jax.numpy as jnp
from jax import lax
from jax.experimental import pallas as pl
from jax.experimental.pallas import tpu as pltpu
from jax.experimental.pallas import tpu_sc as plsc

_VOCAB = 1000000
_EMBED = 64
_BATCH = 16384
_PRED = 20

_NC = 2
_NS = 16
_NW = _NC * _NS
_ROWS_W = _BATCH // _NW
_CHUNK = 32
_NCHUNK = _ROWS_W // _CHUNK
_IDX_G = 128
_G_PER_CHUNK = _CHUNK * _PRED // _IDX_G

_LN2 = 0.6931471805599453
_NEG = -1e30


def _vlog(x):
    bits = lax.bitcast_convert_type(x, jnp.int32)
    e = ((bits >> 23) & 0xFF) - 127
    m = lax.bitcast_convert_type(
        (bits & 0x007FFFFF) | 0x3F800000, jnp.float32)
    big = m > 1.4142135381698608
    m = jnp.where(big, m * 0.5, m)
    ef = (e + big.astype(jnp.int32)).astype(jnp.float32)
    t = m - 1.0
    z = t / (t + 2.0)
    z2 = z * z
    s = 2.0 * z * (1.0 + z2 * (1.0 / 3.0 + z2 * (0.2 + z2 * (1.0 / 7.0))))
    return ef * _LN2 + s


def _body(posu, posv, ut, vt, out, uidx, vidx, urows, vrows, accv, sem):
    c = lax.axis_index("c")
    s = lax.axis_index("s")
    wid = s * _NC + c
    lanes = lax.iota(jnp.int32, 16)

    def chunk_body(i, acc):
        row0 = wid * _ROWS_W + i * _CHUNK
        pltpu.sync_copy(posu.at[pl.ds(row0, _CHUNK)], uidx)
        pltpu.sync_copy(posv.at[pl.ds(row0 * _PRED, _CHUNK * _PRED)], vidx)
        cps = [pltpu.async_copy(ut.at[uidx], urows, sem)]
        for g in range(_G_PER_CHUNK):
            cps.append(pltpu.async_copy(
                vt.at[vidx.at[pl.ds(g * _IDX_G, _IDX_G)]],
                vrows.at[pl.ds(g * _IDX_G, _IDX_G)], sem))
        for cp in cps:
            cp.wait()

        def row_body(r, acc2):
            u = [urows[r, pl.ds(16 * k, 16)] for k in range(4)]
            d0 = None
            v0 = jnp.zeros((16,), jnp.float32)
            v1 = jnp.full((16,), _NEG, jnp.float32)
            for p in range(_PRED):
                w = [vrows[r * _PRED + p, pl.ds(16 * k, 16)] for k in range(4)]
                part = (u[0] * w[0] + u[1] * w[1]) + (u[2] * w[2] + u[3] * w[3])
                dp = jnp.sum(part)
                if p == 0:
                    d0 = dp
                if p < 16:
                    v0 = jnp.where(lanes == p, dp, v0)
                else:
                    v1 = jnp.where(lanes == (p - 16), dp, v1)
            mx = jnp.max(jnp.maximum(v0, v1))
            ssum = jnp.sum(jnp.exp(v0 - mx) + jnp.exp(v1 - mx))
            sv = jnp.broadcast_to(ssum, (16,))
            return acc2 + (_vlog(sv) + (mx - d0))

        return lax.fori_loop(0, _CHUNK, row_body, acc)

    acc = lax.fori_loop(0, _NCHUNK, chunk_body, jnp.zeros((16,), jnp.float32))
    accv[...] = acc
    pltpu.sync_copy(accv, out.at[wid])


@jax.jit
def kernel(pos_u, pos_neg_v, u_table, v_table):
    posu = pos_u.reshape(_BATCH)
    posv = pos_neg_v.reshape(_BATCH * _PRED)
    mesh = plsc.VectorSubcoreMesh(core_axis_name="c", subcore_axis_name="s")
    f = functools.partial(
        pl.kernel,
        out_type=jax.ShapeDtypeStruct((_NW, 16), jnp.float32),
        mesh=mesh,
        scratch_types=[
            pltpu.VMEM((_CHUNK,), jnp.int32),
            pltpu.VMEM((_CHUNK * _PRED,), jnp.int32),
            pltpu.VMEM((_CHUNK, _EMBED), jnp.float32),
            pltpu.VMEM((_CHUNK * _PRED, _EMBED), jnp.float32),
            pltpu.VMEM((16,), jnp.float32),
            pltpu.SemaphoreType.DMA,
        ],
        compiler_params=pltpu.CompilerParams(
            needs_layout_passes=False, use_tc_tiling_on_sc=False),
    )(_body)
    partials = f(posu, posv, u_table, v_table)
    return jnp.sum(partials[:, 0]) / _BATCH
